# baseline (speedup 1.0000x reference)
import functools

import jax
import jax.numpy as jnp
from jax import lax
from jax.experimental import pallas as pl
from jax.experimental.pallas import tpu as pltpu
from jax.experimental.pallas import tpu_sc as plsc

B = 16384
E = 32
H1 = E // 2
H2 = E // 4
H3 = E // 8
N = 1000000

NC, NS = 2, 16
NW = NC * NS
BPW = B // NW
CHUNK = 128
NCH = BPW // CHUNK

@functools.cache
def _build_sc_gather4():
    mesh = plsc.VectorSubcoreMesh(core_axis_name="c", subcore_axis_name="s")

    @functools.partial(
        pl.kernel,
        mesh=mesh,
        out_type=[jax.ShapeDtypeStruct((B, E), jnp.float32)] * 4,
        scratch_types=[
            pltpu.VMEM((NCH, CHUNK), jnp.int32),
            pltpu.VMEM((NCH, CHUNK), jnp.int32),
            pltpu.VMEM((BPW, E), jnp.float32),
            pltpu.VMEM((BPW, E), jnp.float32),
            pltpu.VMEM((BPW, E), jnp.float32),
            pltpu.VMEM((BPW, E), jnp.float32),
            pltpu.SemaphoreType.DMA,
        ],
        compiler_params=pltpu.CompilerParams(use_tc_tiling_on_sc=False),
    )
    def _sc_gather4(user_h, item_h, t_gu, t_gi, t_mu, t_mi,
                    o_gu, o_gi, o_mu, o_mi,
                    idx_u, idx_i, r_gu, r_gi, r_mu, r_mi, sem):
        wid = lax.axis_index("s") * NC + lax.axis_index("c")
        base = wid * BPW
        for c in range(NCH):
            pltpu.sync_copy(user_h.at[pl.ds(base + c * CHUNK, CHUNK)],
                            idx_u.at[c])
            pltpu.sync_copy(item_h.at[pl.ds(base + c * CHUNK, CHUNK)],
                            idx_i.at[c])
        copies = []
        for c in range(NCH):
            sl = pl.ds(c * CHUNK, CHUNK)
            copies.append(pltpu.async_copy(t_gu.at[idx_u.at[c]], r_gu.at[sl], sem))
            copies.append(pltpu.async_copy(t_gi.at[idx_i.at[c]], r_gi.at[sl], sem))
            copies.append(pltpu.async_copy(t_mu.at[idx_u.at[c]], r_mu.at[sl], sem))
            copies.append(pltpu.async_copy(t_mi.at[idx_i.at[c]], r_mi.at[sl], sem))
        for cp in copies:
            cp.wait()
        pltpu.sync_copy(r_gu, o_gu.at[pl.ds(base, BPW)])
        pltpu.sync_copy(r_gi, o_gi.at[pl.ds(base, BPW)])
        pltpu.sync_copy(r_mu, o_mu.at[pl.ds(base, BPW)])
        pltpu.sync_copy(r_mi, o_mi.at[pl.ds(base, BPW)])

    return _sc_gather4


TCOL = 8192
TGRID = -(-N // TCOL)


def _tr_body(a, b, c, d, oa, ob, oc, od):
    oa[...] = a[...].T
    ob[...] = b[...].T
    oc[...] = c[...].T
    od[...] = d[...].T


def _tc_transpose4(a, b, c, d):
    ispec = pl.BlockSpec((E, TCOL), lambda i: (0, i))
    ospec = pl.BlockSpec((TCOL, E), lambda i: (i, 0))
    return pl.pallas_call(
        _tr_body,
        grid=(TGRID,),
        in_specs=[ispec] * 4,
        out_specs=[ospec] * 4,
        out_shape=[jax.ShapeDtypeStruct((N, E), jnp.float32)] * 4,
    )(a, b, c, d)


BLK = 2048


def _tc_body(gu, gi, mu, mi, w1a, w1b, b1, w2, b2, w3, b3, wog, wom, bo, out):
    h = jnp.dot(mu[...], w1a[...], preferred_element_type=jnp.float32)
    h = h + jnp.dot(mi[...], w1b[...], preferred_element_type=jnp.float32)
    h = jnp.maximum(h + b1[...], 0.0)
    h = jnp.maximum(
        jnp.dot(h, w2[...], preferred_element_type=jnp.float32) + b2[...], 0.0)
    pm = jnp.dot(h, w3[...], preferred_element_type=jnp.float32) + b3[...]
    pg = gu[...] * gi[...]
    logit = (jnp.sum(pg * wog[...], axis=1, keepdims=True)
             + jnp.sum(pm * wom[...], axis=1, keepdims=True) + bo[...])
    out[...] = jax.nn.sigmoid(logit)


def _tc_dense(gu, gi, mu, mi, w1a, w1b, b1, w2, b2, w3, b3, wog, wom, bo):
    full = lambda shape: pl.BlockSpec(shape, lambda i: (0, 0))
    return pl.pallas_call(
        _tc_body,
        grid=(B // BLK,),
        in_specs=[
            pl.BlockSpec((BLK, E), lambda i: (i, 0)),
            pl.BlockSpec((BLK, E), lambda i: (i, 0)),
            pl.BlockSpec((BLK, E), lambda i: (i, 0)),
            pl.BlockSpec((BLK, E), lambda i: (i, 0)),
            full((E, H1)), full((E, H1)), full((1, H1)),
            full((H1, H2)), full((1, H2)),
            full((H2, H3)), full((1, H3)),
            full((1, E)), full((1, H3)), full((1, 1)),
        ],
        out_specs=pl.BlockSpec((BLK, 1), lambda i: (i, 0)),
        out_shape=jax.ShapeDtypeStruct((B, 1), jnp.float32),
    )(gu, gi, mu, mi, w1a, w1b, b1, w2, b2, w3, b3, wog, wom, bo)


def kernel(user, item, gmf_user_w, gmf_item_w, mlp_user_w, mlp_item_w,
           W1, b1, W2, b2, W3, b3, Wo, bo):
    t_gu, t_gi, t_mu, t_mi = _tc_transpose4(
        gmf_user_w.T, gmf_item_w.T, mlp_user_w.T, mlp_item_w.T)
    gu, gi, mu, mi = _build_sc_gather4()(user, item, t_gu, t_gi, t_mu, t_mi)
    w1a, w1b = W1[:E], W1[E:]
    wog = Wo[:E].reshape(1, E)
    wom = Wo[E:].reshape(1, H3)
    return _tc_dense(gu, gi, mu, mi, w1a, w1b,
                     b1.reshape(1, H1), W2, b2.reshape(1, H2),
                     W3, b3.reshape(1, H3), wog, wom, bo.reshape(1, 1))


# speedup vs baseline: 1.9443x; 1.9443x over previous
import functools

import jax
import jax.numpy as jnp
from jax import lax
from jax.experimental import pallas as pl
from jax.experimental.pallas import tpu as pltpu
from jax.experimental.pallas import tpu_sc as plsc

B = 16384
E = 32
H1 = E // 2
H2 = E // 4
H3 = E // 8
N = 1000000

TQ = 262144
QSH = 18
TC2 = 2048
TG2 = TQ // TC2

NC, NS = 2, 16
NW = NC * NS
BPW = B // NW
CHUNK = 128
NCH = BPW // CHUNK


def _tr_body(*refs):
    ins, outs = refs[:16], refs[16:]
    for t in range(4):
        outs[t][...] = jnp.concatenate(
            [x[...].T for x in ins[4 * t:4 * t + 4]], axis=1)


NBLK = -(-N // TC2)


def _tc_pack4(a, b, c, d):
    in_specs = [
        pl.BlockSpec(
            (E, TC2), lambda i, k=k: (0, jnp.minimum(k * TG2 + i, NBLK - 1)))
        for _ in range(4) for k in range(4)
    ]
    return pl.pallas_call(
        _tr_body,
        grid=(TG2,),
        in_specs=in_specs,
        out_specs=[pl.BlockSpec((TC2, 4 * E), lambda i: (i, 0))] * 4,
        out_shape=[jax.ShapeDtypeStruct((TQ, 4 * E), jnp.float32)] * 4,
    )(a, a, a, a, b, b, b, b, c, c, c, c, d, d, d, d)


@functools.cache
def _build_sc_gather4():
    mesh = plsc.VectorSubcoreMesh(core_axis_name="c", subcore_axis_name="s")

    @functools.partial(
        pl.kernel,
        mesh=mesh,
        out_type=[jax.ShapeDtypeStruct((B, 4 * E), jnp.float32)] * 4,
        scratch_types=[
            pltpu.VMEM((NCH, CHUNK), jnp.int32),
            pltpu.VMEM((NCH, CHUNK), jnp.int32),
            pltpu.VMEM((CHUNK, 4 * E), jnp.float32),
            pltpu.VMEM((CHUNK, 4 * E), jnp.float32),
            pltpu.VMEM((CHUNK, 4 * E), jnp.float32),
            pltpu.VMEM((CHUNK, 4 * E), jnp.float32),
            pltpu.SemaphoreType.DMA,
        ],
        compiler_params=pltpu.CompilerParams(use_tc_tiling_on_sc=False),
    )
    def _sc_gather4(um_h, im_h, t_gu, t_gi, t_mu, t_mi,
                    o_gu, o_gi, o_mu, o_mi,
                    idx_u, idx_i, r_gu, r_gi, r_mu, r_mi, sem):
        wid = lax.axis_index("s") * NC + lax.axis_index("c")
        base = wid * BPW
        for c in range(NCH):
            pltpu.sync_copy(um_h.at[pl.ds(base + c * CHUNK, CHUNK)],
                            idx_u.at[c])
            pltpu.sync_copy(im_h.at[pl.ds(base + c * CHUNK, CHUNK)],
                            idx_i.at[c])
        for c in range(NCH):
            copies = [
                pltpu.async_copy(t_gu.at[idx_u.at[c]], r_gu, sem),
                pltpu.async_copy(t_gi.at[idx_i.at[c]], r_gi, sem),
                pltpu.async_copy(t_mu.at[idx_u.at[c]], r_mu, sem),
                pltpu.async_copy(t_mi.at[idx_i.at[c]], r_mi, sem),
            ]
            for cp in copies:
                cp.wait()
            dst = pl.ds(base + c * CHUNK, CHUNK)
            pltpu.sync_copy(r_gu, o_gu.at[dst])
            pltpu.sync_copy(r_gi, o_gi.at[dst])
            pltpu.sync_copy(r_mu, o_mu.at[dst])
            pltpu.sync_copy(r_mi, o_mi.at[dst])

    return _sc_gather4


BLK = 2048


def _quarter(rows, q):
    e = rows[:, 0:E]
    for k in range(1, 4):
        e = jnp.where(q == k, rows[:, E * k:E * (k + 1)], e)
    return e


def _tc_body(qu, qi, gu, gi, mu, mi,
             w1a, w1b, b1, w2, b2, w3, b3, wog, wom, bo, out):
    q_u = qu[...]
    q_i = qi[...]
    guv = _quarter(gu[...], q_u)
    giv = _quarter(gi[...], q_i)
    muv = _quarter(mu[...], q_u)
    miv = _quarter(mi[...], q_i)
    h = jnp.dot(muv, w1a[...], preferred_element_type=jnp.float32)
    h = h + jnp.dot(miv, w1b[...], preferred_element_type=jnp.float32)
    h = jnp.maximum(h + b1[...], 0.0)
    h = jnp.maximum(
        jnp.dot(h, w2[...], preferred_element_type=jnp.float32) + b2[...], 0.0)
    pm = jnp.dot(h, w3[...], preferred_element_type=jnp.float32) + b3[...]
    pg = guv * giv
    logit = (jnp.sum(pg * wog[...], axis=1, keepdims=True)
             + jnp.sum(pm * wom[...], axis=1, keepdims=True) + bo[...])
    out[...] = jax.nn.sigmoid(logit)


def _tc_dense(qu, qi, gu, gi, mu, mi,
              w1a, w1b, b1, w2, b2, w3, b3, wog, wom, bo):
    full = lambda shape: pl.BlockSpec(shape, lambda i: (0, 0))
    return pl.pallas_call(
        _tc_body,
        grid=(B // BLK,),
        in_specs=[
            pl.BlockSpec((BLK, 1), lambda i: (i, 0)),
            pl.BlockSpec((BLK, 1), lambda i: (i, 0)),
            pl.BlockSpec((BLK, 4 * E), lambda i: (i, 0)),
            pl.BlockSpec((BLK, 4 * E), lambda i: (i, 0)),
            pl.BlockSpec((BLK, 4 * E), lambda i: (i, 0)),
            pl.BlockSpec((BLK, 4 * E), lambda i: (i, 0)),
            full((E, H1)), full((E, H1)), full((1, H1)),
            full((H1, H2)), full((1, H2)),
            full((H2, H3)), full((1, H3)),
            full((1, E)), full((1, H3)), full((1, 1)),
        ],
        out_specs=pl.BlockSpec((BLK, 1), lambda i: (i, 0)),
        out_shape=jax.ShapeDtypeStruct((B, 1), jnp.float32),
    )(qu, qi, gu, gi, mu, mi, w1a, w1b, b1, w2, b2, w3, b3, wog, wom, bo)


def kernel(user, item, gmf_user_w, gmf_item_w, mlp_user_w, mlp_item_w,
           W1, b1, W2, b2, W3, b3, Wo, bo):
    um = user & (TQ - 1)
    im = item & (TQ - 1)
    qu = (user >> QSH).reshape(B, 1)
    qi = (item >> QSH).reshape(B, 1)
    t_gu, t_gi, t_mu, t_mi = _tc_pack4(
        gmf_user_w.T, gmf_item_w.T, mlp_user_w.T, mlp_item_w.T)
    gu, gi, mu, mi = _build_sc_gather4()(um, im, t_gu, t_gi, t_mu, t_mi)
    w1a, w1b = W1[:E], W1[E:]
    wog = Wo[:E].reshape(1, E)
    wom = Wo[E:].reshape(1, H3)
    return _tc_dense(qu, qi, gu, gi, mu, mi, w1a, w1b,
                     b1.reshape(1, H1), W2, b2.reshape(1, H2),
                     W3, b3.reshape(1, H3), wog, wom, bo.reshape(1, 1))


# speedup vs baseline: 3.7179x; 1.9122x over previous
import functools

import jax
import jax.numpy as jnp
from jax import lax
from jax.experimental import pallas as pl
from jax.experimental.pallas import tpu as pltpu
from jax.experimental.pallas import tpu_sc as plsc

B = 16384
E = 32
H1 = E // 2
H2 = E // 4
H3 = E // 8
N = 1000000

TC2 = 2048
TG2 = -(-N // TC2)

NC, NS = 2, 16
NW = NC * NS
BPW = B // NW
CHUNK = 128
NCH = BPW // CHUNK


def _pack_body(a, b, c, d, out):
    x = jnp.concatenate([a[...], b[...], c[...], d[...]], axis=0)
    out[...] = x.T


def _tc_pack4(a, b, c, d):
    return pl.pallas_call(
        _pack_body,
        grid=(TG2,),
        in_specs=[pl.BlockSpec((E, TC2), lambda i: (0, i))] * 4,
        out_specs=pl.BlockSpec((TC2, 4 * E), lambda i: (i, 0)),
        out_shape=jax.ShapeDtypeStruct((N, 4 * E), jnp.float32),
    )(a, b, c, d)


@functools.cache
def _build_sc_gather2():
    mesh = plsc.VectorSubcoreMesh(core_axis_name="c", subcore_axis_name="s")

    @functools.partial(
        pl.kernel,
        mesh=mesh,
        out_type=[jax.ShapeDtypeStruct((B, 4 * E), jnp.float32)] * 2,
        scratch_types=[
            pltpu.VMEM((NCH, CHUNK), jnp.int32),
            pltpu.VMEM((NCH, CHUNK), jnp.int32),
            pltpu.VMEM((CHUNK, 4 * E), jnp.float32),
            pltpu.VMEM((CHUNK, 4 * E), jnp.float32),
            pltpu.SemaphoreType.DMA,
        ],
        compiler_params=pltpu.CompilerParams(use_tc_tiling_on_sc=False),
    )
    def _sc_gather2(user_h, item_h, table,
                    o_u, o_i,
                    idx_u, idx_i, r_u, r_i, sem):
        wid = lax.axis_index("s") * NC + lax.axis_index("c")
        base = wid * BPW
        for c in range(NCH):
            pltpu.sync_copy(user_h.at[pl.ds(base + c * CHUNK, CHUNK)],
                            idx_u.at[c])
            pltpu.sync_copy(item_h.at[pl.ds(base + c * CHUNK, CHUNK)],
                            idx_i.at[c])
        for c in range(NCH):
            cp_u = pltpu.async_copy(table.at[idx_u.at[c]], r_u, sem)
            cp_i = pltpu.async_copy(table.at[idx_i.at[c]], r_i, sem)
            cp_u.wait()
            cp_i.wait()
            dst = pl.ds(base + c * CHUNK, CHUNK)
            pltpu.sync_copy(r_u, o_u.at[dst])
            pltpu.sync_copy(r_i, o_i.at[dst])

    return _sc_gather2


BLK = 2048


def _tc_body(ur, ir, w1a, w1b, b1, w2, b2, w3, b3, wog, wom, bo, out):
    u_row = ur[...]
    i_row = ir[...]
    guv = u_row[:, 0:E]
    giv = i_row[:, E:2 * E]
    muv = u_row[:, 2 * E:3 * E]
    miv = i_row[:, 3 * E:4 * E]
    h = jnp.dot(muv, w1a[...], preferred_element_type=jnp.float32)
    h = h + jnp.dot(miv, w1b[...], preferred_element_type=jnp.float32)
    h = jnp.maximum(h + b1[...], 0.0)
    h = jnp.maximum(
        jnp.dot(h, w2[...], preferred_element_type=jnp.float32) + b2[...], 0.0)
    pm = jnp.dot(h, w3[...], preferred_element_type=jnp.float32) + b3[...]
    pg = guv * giv
    logit = (jnp.sum(pg * wog[...], axis=1, keepdims=True)
             + jnp.sum(pm * wom[...], axis=1, keepdims=True) + bo[...])
    out[...] = jax.nn.sigmoid(logit)


def _tc_dense(ur, ir, w1a, w1b, b1, w2, b2, w3, b3, wog, wom, bo):
    full = lambda shape: pl.BlockSpec(shape, lambda i: (0, 0))
    return pl.pallas_call(
        _tc_body,
        grid=(B // BLK,),
        in_specs=[
            pl.BlockSpec((BLK, 4 * E), lambda i: (i, 0)),
            pl.BlockSpec((BLK, 4 * E), lambda i: (i, 0)),
            full((E, H1)), full((E, H1)), full((1, H1)),
            full((H1, H2)), full((1, H2)),
            full((H2, H3)), full((1, H3)),
            full((1, E)), full((1, H3)), full((1, 1)),
        ],
        out_specs=pl.BlockSpec((BLK, 1), lambda i: (i, 0)),
        out_shape=jax.ShapeDtypeStruct((B, 1), jnp.float32),
    )(ur, ir, w1a, w1b, b1, w2, b2, w3, b3, wog, wom, bo)


def kernel(user, item, gmf_user_w, gmf_item_w, mlp_user_w, mlp_item_w,
           W1, b1, W2, b2, W3, b3, Wo, bo):
    packed = _tc_pack4(
        gmf_user_w.T, gmf_item_w.T, mlp_user_w.T, mlp_item_w.T)
    u_rows, i_rows = _build_sc_gather2()(user, item, packed)
    w1a, w1b = W1[:E], W1[E:]
    wog = Wo[:E].reshape(1, E)
    wom = Wo[E:].reshape(1, H3)
    return _tc_dense(u_rows, i_rows, w1a, w1b,
                     b1.reshape(1, H1), W2, b2.reshape(1, H2),
                     W3, b3.reshape(1, H3), wog, wom, bo.reshape(1, 1))


# speedup vs baseline: 4.9380x; 1.3282x over previous
import functools

import jax
import jax.numpy as jnp
from jax import lax
from jax.experimental import pallas as pl
from jax.experimental.pallas import tpu as pltpu
from jax.experimental.pallas import tpu_sc as plsc

B = 16384
E = 32
H1 = E // 2
H2 = E // 4
H3 = E // 8
N = 1000000

TC2 = 4096
TG2 = -(-N // TC2)

NC, NS = 2, 16
NW = NC * NS
BPW = B // NW
CHUNK = 128
NCH = BPW // CHUNK


def _pack_body(a, b, c, d, out):
    x = jnp.concatenate([a[...], b[...], c[...], d[...]], axis=0)
    out[...] = x.T


def _tc_pack4(a, b, c, d):
    return pl.pallas_call(
        _pack_body,
        grid=(TG2,),
        in_specs=[pl.BlockSpec((E, TC2), lambda i: (0, i))] * 4,
        out_specs=pl.BlockSpec((TC2, 4 * E), lambda i: (i, 0)),
        out_shape=jax.ShapeDtypeStruct((N, 4 * E), jnp.float32),
    )(a, b, c, d)


@functools.cache
def _build_sc_gather2():
    mesh = plsc.VectorSubcoreMesh(core_axis_name="c", subcore_axis_name="s")

    @functools.partial(
        pl.kernel,
        mesh=mesh,
        out_type=[jax.ShapeDtypeStruct((B, 4 * E), jnp.float32)] * 2,
        scratch_types=[
            pltpu.VMEM((NCH, CHUNK), jnp.int32),
            pltpu.VMEM((NCH, CHUNK), jnp.int32),
            pltpu.VMEM((CHUNK, 4 * E), jnp.float32),
            pltpu.VMEM((CHUNK, 4 * E), jnp.float32),
            pltpu.SemaphoreType.DMA,
        ],
        compiler_params=pltpu.CompilerParams(use_tc_tiling_on_sc=False),
    )
    def _sc_gather2(user_h, item_h, table,
                    o_u, o_i,
                    idx_u, idx_i, r_u, r_i, sem):
        wid = lax.axis_index("s") * NC + lax.axis_index("c")
        base = wid * BPW
        for c in range(NCH):
            pltpu.sync_copy(user_h.at[pl.ds(base + c * CHUNK, CHUNK)],
                            idx_u.at[c])
            pltpu.sync_copy(item_h.at[pl.ds(base + c * CHUNK, CHUNK)],
                            idx_i.at[c])
        for c in range(NCH):
            cp_u = pltpu.async_copy(table.at[idx_u.at[c]], r_u, sem)
            cp_i = pltpu.async_copy(table.at[idx_i.at[c]], r_i, sem)
            cp_u.wait()
            cp_i.wait()
            dst = pl.ds(base + c * CHUNK, CHUNK)
            pltpu.sync_copy(r_u, o_u.at[dst])
            pltpu.sync_copy(r_i, o_i.at[dst])

    return _sc_gather2


BLK = 2048


def _tc_body(ur, ir, w1a, w1b, b1, w2, b2, w3, b3, wog, wom, bo, out):
    u_row = ur[...]
    i_row = ir[...]
    guv = u_row[:, 0:E]
    giv = i_row[:, E:2 * E]
    muv = u_row[:, 2 * E:3 * E]
    miv = i_row[:, 3 * E:4 * E]
    h = jnp.dot(muv, w1a[...], preferred_element_type=jnp.float32)
    h = h + jnp.dot(miv, w1b[...], preferred_element_type=jnp.float32)
    h = jnp.maximum(h + b1[...], 0.0)
    h = jnp.maximum(
        jnp.dot(h, w2[...], preferred_element_type=jnp.float32) + b2[...], 0.0)
    pm = jnp.dot(h, w3[...], preferred_element_type=jnp.float32) + b3[...]
    pg = guv * giv
    logit = (jnp.sum(pg * wog[...], axis=1, keepdims=True)
             + jnp.sum(pm * wom[...], axis=1, keepdims=True) + bo[...])
    out[...] = jax.nn.sigmoid(logit)


def _tc_dense(ur, ir, w1a, w1b, b1, w2, b2, w3, b3, wog, wom, bo):
    full = lambda shape: pl.BlockSpec(shape, lambda i: (0, 0))
    return pl.pallas_call(
        _tc_body,
        grid=(B // BLK,),
        in_specs=[
            pl.BlockSpec((BLK, 4 * E), lambda i: (i, 0)),
            pl.BlockSpec((BLK, 4 * E), lambda i: (i, 0)),
            full((E, H1)), full((E, H1)), full((1, H1)),
            full((H1, H2)), full((1, H2)),
            full((H2, H3)), full((1, H3)),
            full((1, E)), full((1, H3)), full((1, 1)),
        ],
        out_specs=pl.BlockSpec((BLK, 1), lambda i: (i, 0)),
        out_shape=jax.ShapeDtypeStruct((B, 1), jnp.float32),
    )(ur, ir, w1a, w1b, b1, w2, b2, w3, b3, wog, wom, bo)


def kernel(user, item, gmf_user_w, gmf_item_w, mlp_user_w, mlp_item_w,
           W1, b1, W2, b2, W3, b3, Wo, bo):
    packed = _tc_pack4(
        gmf_user_w.T, gmf_item_w.T, mlp_user_w.T, mlp_item_w.T)
    u_rows, i_rows = _build_sc_gather2()(user, item, packed)
    w1a, w1b = W1[:E], W1[E:]
    wog = Wo[:E].reshape(1, E)
    wom = Wo[E:].reshape(1, H3)
    return _tc_dense(u_rows, i_rows, w1a, w1b,
                     b1.reshape(1, H1), W2, b2.reshape(1, H2),
                     W3, b3.reshape(1, H3), wog, wom, bo.reshape(1, 1))
